# baseline (speedup 1.0000x reference)
import functools

import jax
import jax.numpy as jnp
from jax import lax
from jax.experimental import pallas as pl
from jax.experimental.pallas import tpu as pltpu
from jax.experimental.pallas import tpu_sc as plsc

D_IN = 768
D_OUT = 768
E = 8
H = 1024
KTOP = 2
T = 2048
BLK = 256
NB = (KTOP * T) // BLK + E
P = NB * BLK
CHUNK = 128
NW = 32
TPW = T // NW
EPS_OUT = 2.220446049250313e-16


def _gating_body(x_ref, wg_ref, pos_ref, gv_ref, meta_ref, loss_ref):
    x = x_ref[...]
    logits = jnp.dot(x, wg_ref[...], preferred_element_type=jnp.float32)
    lane = jax.lax.broadcasted_iota(jnp.int32, (T, E), 1)
    m1 = jnp.max(logits, axis=1, keepdims=True)
    e1 = jnp.min(jnp.where(logits == m1, lane, E), axis=1, keepdims=True)
    masked = jnp.where(lane == e1, -jnp.inf, logits)
    m2 = jnp.max(masked, axis=1, keepdims=True)
    e2 = jnp.min(jnp.where(masked == m2, lane, E), axis=1, keepdims=True)
    u = jnp.exp(m2 - m1)
    g1 = 1.0 / (1.0 + u)
    g2 = u / (1.0 + u)
    gv_ref[...] = jnp.concatenate([g1, g2], axis=1)
    oh1 = lane == e1
    oh2 = lane == e2

    m = jnp.where(oh1 | oh2, 1.0, 0.0)
    r = jax.lax.broadcasted_iota(jnp.int32, (CHUNK, CHUNK), 0)
    c = jax.lax.broadcasted_iota(jnp.int32, (CHUNK, CHUNK), 1)
    ltri = jnp.where(r > c, 1.0, 0.0)
    chunks = []
    carry = jnp.zeros((1, E), jnp.float32)
    for j in range(T // CHUNK):
        mj = m[j * CHUNK : (j + 1) * CHUNK]
        chunks.append(jnp.dot(ltri, mj, preferred_element_type=jnp.float32) + carry)
        carry = carry + jnp.sum(mj, axis=0, keepdims=True)
    excl = jnp.concatenate(chunks, axis=0)

    counts = carry.astype(jnp.int32)
    padded = ((counts + BLK - 1) // BLK) * BLK
    inc = padded
    for sh in (1, 2, 4):
        inc = inc + jnp.concatenate(
            [jnp.zeros((1, sh), jnp.int32), inc[:, :-sh]], axis=1
        )
    seg_start = (inc - padded).astype(jnp.float32)

    def pick(oh):
        return jnp.sum(jnp.where(oh, seg_start + excl, 0.0), axis=1, keepdims=True)

    pos_ref[...] = jnp.concatenate([pick(oh1), pick(oh2)], axis=1).astype(jnp.int32)

    rowb = jax.lax.broadcasted_iota(jnp.int32, (NB, E), 0) * BLK
    blk_e = jnp.minimum(
        jnp.sum((rowb >= inc).astype(jnp.int32), axis=1, keepdims=True), E - 1
    )
    nact = inc[:, E - 1 :] // BLK
    meta_ref[...] = jnp.concatenate([blk_e, nact], axis=0)

    importance = jnp.sum(jnp.where(oh1, g1, 0.0) + jnp.where(oh2, g2, 0.0), axis=0)
    load = jnp.sum(
        oh1.astype(jnp.float32) + jnp.where(oh2 & (g2 > 0.0), 1.0, 0.0), axis=0
    )

    def cv2(v):
        mean = jnp.sum(v) / E
        var = jnp.sum((v - mean) ** 2) / (E - 1)
        return var / (mean * mean + 1e-10)

    loss_ref[...] = ((cv2(importance) + cv2(load)) * 0.01).reshape(1, 1)


def _gating(x, w_gate):
    return pl.pallas_call(
        _gating_body,
        out_shape=[
            jax.ShapeDtypeStruct((T, KTOP), jnp.int32),
            jax.ShapeDtypeStruct((T, KTOP), jnp.float32),
            jax.ShapeDtypeStruct((NB + 1, 1), jnp.int32),
            jax.ShapeDtypeStruct((1, 1), jnp.float32),
        ],
    )(x, w_gate)


_SC_MESH = plsc.VectorSubcoreMesh(core_axis_name="c", subcore_axis_name="s")


@functools.partial(
    pl.kernel,
    mesh=_SC_MESH,
    out_type=[
        jax.ShapeDtypeStruct((P, D_IN), jnp.float32),
        jax.ShapeDtypeStruct((P,), jnp.float32),
    ],
    scratch_types=[
        pltpu.VMEM((TPW, D_IN), jnp.float32),
        pltpu.VMEM((TPW,), jnp.int32),
        pltpu.VMEM((TPW,), jnp.int32),
        pltpu.VMEM((TPW,), jnp.float32),
        pltpu.VMEM((TPW,), jnp.float32),
        pltpu.SemaphoreType.DMA,
    ],
)
def _dispatch(x_hbm, p0_hbm, p1_hbm, g0_hbm, g1_hbm, xs_hbm, gp_hbm,
              rows_v, i0_v, i1_v, ga_v, gb_v, sem):
    wid = lax.axis_index("s") * 2 + lax.axis_index("c")
    base = wid * TPW
    pltpu.sync_copy(x_hbm.at[pl.ds(base, TPW)], rows_v)
    pltpu.sync_copy(p0_hbm.at[pl.ds(base, TPW)], i0_v)
    pltpu.sync_copy(p1_hbm.at[pl.ds(base, TPW)], i1_v)
    pltpu.sync_copy(g0_hbm.at[pl.ds(base, TPW)], ga_v)
    pltpu.sync_copy(g1_hbm.at[pl.ds(base, TPW)], gb_v)
    c1 = pltpu.async_copy(rows_v, xs_hbm.at[i0_v], sem)
    c2 = pltpu.async_copy(rows_v, xs_hbm.at[i1_v], sem)
    c3 = pltpu.async_copy(ga_v, gp_hbm.at[i0_v], sem)
    c4 = pltpu.async_copy(gb_v, gp_hbm.at[i1_v], sem)
    c1.wait()
    c2.wait()
    c3.wait()
    c4.wait()


@functools.partial(
    pl.kernel,
    mesh=_SC_MESH,
    out_type=jax.ShapeDtypeStruct((T, D_OUT), jnp.float32),
    scratch_types=[
        pltpu.VMEM((TPW, D_OUT), jnp.float32),
        pltpu.VMEM((TPW, D_OUT), jnp.float32),
        pltpu.VMEM((TPW,), jnp.int32),
        pltpu.VMEM((TPW,), jnp.int32),
        pltpu.SemaphoreType.DMA,
    ],
)
def _combine(r_hbm, p0_hbm, p1_hbm, out_hbm, a_v, b_v, i0_v, i1_v, sem):
    wid = lax.axis_index("s") * 2 + lax.axis_index("c")
    base = wid * TPW
    pltpu.sync_copy(p0_hbm.at[pl.ds(base, TPW)], i0_v)
    pltpu.sync_copy(p1_hbm.at[pl.ds(base, TPW)], i1_v)
    c1 = pltpu.async_copy(r_hbm.at[i0_v], a_v, sem)
    c2 = pltpu.async_copy(r_hbm.at[i1_v], b_v, sem)
    c1.wait()
    c2.wait()
    eps = jnp.full((16,), EPS_OUT, jnp.float32)

    @plsc.parallel_loop(0, TPW, 1, unroll=4)
    def _row(t):
        for cidx in range(D_OUT // 16):
            sl = pl.ds(cidx * 16, 16)
            v = a_v[t, sl] + b_v[t, sl]
            a_v[t, sl] = jnp.where(v == 0.0, eps, v)
    pltpu.sync_copy(a_v, out_hbm.at[pl.ds(base, TPW)])


def _mlp_body(meta_ref, xs_ref, w1_ref, b1_ref, w2_ref, b2_ref, gp_ref, out_ref):
    b = pl.program_id(0)
    nact = meta_ref[NB, 0]

    @pl.when(b < nact)
    def _():
        xb = xs_ref[...]
        h = jnp.maximum(
            jnp.dot(xb, w1_ref[0], preferred_element_type=jnp.float32) + b1_ref[0],
            0.0,
        )
        o = jnp.dot(h, w2_ref[0], preferred_element_type=jnp.float32) + b2_ref[0]
        out_ref[...] = o * gp_ref[...]


def _grouped_mlp(meta, xs, W1, b1, W2, b2, gate_pad):
    grid_spec = pltpu.PrefetchScalarGridSpec(
        num_scalar_prefetch=1,
        grid=(NB,),
        in_specs=[
            pl.BlockSpec((BLK, D_IN), lambda b, m: (b, 0)),
            pl.BlockSpec((1, D_IN, H), lambda b, m: (m[b, 0], 0, 0)),
            pl.BlockSpec((1, 1, H), lambda b, m: (m[b, 0], 0, 0)),
            pl.BlockSpec((1, H, D_OUT), lambda b, m: (m[b, 0], 0, 0)),
            pl.BlockSpec((1, 1, D_OUT), lambda b, m: (m[b, 0], 0, 0)),
            pl.BlockSpec((BLK, 1), lambda b, m: (b, 0)),
        ],
        out_specs=pl.BlockSpec((BLK, D_OUT), lambda b, m: (b, 0)),
    )
    return pl.pallas_call(
        _mlp_body,
        grid_spec=grid_spec,
        out_shape=jax.ShapeDtypeStruct((P, D_OUT), jnp.float32),
    )(meta, xs, W1, b1.reshape(E, 1, H), W2, b2.reshape(E, 1, D_OUT),
      gate_pad.reshape(P, 1))


def kernel(x, w_gate, W1, b1, W2, b2):
    pos, gv, meta, loss = _gating(x, w_gate)
    p0 = pos[:, 0]
    p1 = pos[:, 1]
    xs, gate_pad = _dispatch(x, p0, p1, gv[:, 0], gv[:, 1])
    out_sorted = _grouped_mlp(meta, xs, W1, b1, W2, b2, gate_pad)
    combined = _combine(out_sorted, p0, p1)
    return combined, loss.reshape(())


# speedup vs baseline: 1.3730x; 1.3730x over previous
import functools

import jax
import jax.numpy as jnp
from jax import lax
from jax.experimental import pallas as pl
from jax.experimental.pallas import tpu as pltpu
from jax.experimental.pallas import tpu_sc as plsc

D_IN = 768
D_OUT = 768
E = 8
H = 1024
KTOP = 2
T = 2048
BLK = 256
NB = (KTOP * T) // BLK + E
P = NB * BLK
CHUNK = 128
NW = 32
TPW = T // NW
EPS_OUT = 2.220446049250313e-16


def _gating_body(x_ref, wg_ref, pos_ref, g0r_ref, g1r_ref, meta_ref, loss_ref):
    x = x_ref[...]
    logits = jnp.dot(x, wg_ref[...], preferred_element_type=jnp.float32)
    lane = jax.lax.broadcasted_iota(jnp.int32, (T, E), 1)
    m1 = jnp.max(logits, axis=1, keepdims=True)
    e1 = jnp.min(jnp.where(logits == m1, lane, E), axis=1, keepdims=True)
    masked = jnp.where(lane == e1, -jnp.inf, logits)
    m2 = jnp.max(masked, axis=1, keepdims=True)
    e2 = jnp.min(jnp.where(masked == m2, lane, E), axis=1, keepdims=True)
    u = jnp.exp(m2 - m1)
    g1 = 1.0 / (1.0 + u)
    g2 = u / (1.0 + u)
    g0r_ref[...] = jnp.broadcast_to(g1, (T, 16))
    g1r_ref[...] = jnp.broadcast_to(g2, (T, 16))
    oh1 = lane == e1
    oh2 = lane == e2

    m = jnp.where(oh1 | oh2, 1.0, 0.0)
    r = jax.lax.broadcasted_iota(jnp.int32, (CHUNK, CHUNK), 0)
    c = jax.lax.broadcasted_iota(jnp.int32, (CHUNK, CHUNK), 1)
    ltri = jnp.where(r > c, 1.0, 0.0)
    chunks = []
    carry = jnp.zeros((1, E), jnp.float32)
    for j in range(T // CHUNK):
        mj = m[j * CHUNK : (j + 1) * CHUNK]
        chunks.append(jnp.dot(ltri, mj, preferred_element_type=jnp.float32) + carry)
        carry = carry + jnp.sum(mj, axis=0, keepdims=True)
    excl = jnp.concatenate(chunks, axis=0)

    counts = carry.astype(jnp.int32)
    padded = ((counts + BLK - 1) // BLK) * BLK
    inc = padded
    for sh in (1, 2, 4):
        inc = inc + jnp.concatenate(
            [jnp.zeros((1, sh), jnp.int32), inc[:, :-sh]], axis=1
        )
    seg_start = (inc - padded).astype(jnp.float32)

    def pick(oh):
        return jnp.sum(jnp.where(oh, seg_start + excl, 0.0), axis=1, keepdims=True)

    pos_ref[...] = jnp.concatenate([pick(oh1), pick(oh2)], axis=1).astype(jnp.int32)

    rowb = jax.lax.broadcasted_iota(jnp.int32, (NB, E), 0) * BLK
    blk_e = jnp.minimum(
        jnp.sum((rowb >= inc).astype(jnp.int32), axis=1, keepdims=True), E - 1
    )
    nact = inc[:, E - 1 :] // BLK
    meta_ref[...] = jnp.concatenate([blk_e, nact], axis=0)

    importance = jnp.sum(jnp.where(oh1, g1, 0.0) + jnp.where(oh2, g2, 0.0), axis=0)
    load = jnp.sum(
        oh1.astype(jnp.float32) + jnp.where(oh2 & (g2 > 0.0), 1.0, 0.0), axis=0
    )

    def cv2(v):
        mean = jnp.sum(v) / E
        var = jnp.sum((v - mean) ** 2) / (E - 1)
        return var / (mean * mean + 1e-10)

    loss_ref[...] = ((cv2(importance) + cv2(load)) * 0.01).reshape(1, 1)


def _gating(x, w_gate):
    return pl.pallas_call(
        _gating_body,
        out_shape=[
            jax.ShapeDtypeStruct((T, KTOP), jnp.int32),
            jax.ShapeDtypeStruct((T, 16), jnp.float32),
            jax.ShapeDtypeStruct((T, 16), jnp.float32),
            jax.ShapeDtypeStruct((NB + 1, 1), jnp.int32),
            jax.ShapeDtypeStruct((1, 1), jnp.float32),
        ],
    )(x, w_gate)


_SC_MESH = plsc.VectorSubcoreMesh(core_axis_name="c", subcore_axis_name="s")


@functools.partial(
    pl.kernel,
    mesh=_SC_MESH,
    out_type=jax.ShapeDtypeStruct((P, D_IN), jnp.float32),
    scratch_types=[
        pltpu.VMEM((TPW, D_IN), jnp.float32),
        pltpu.VMEM((TPW,), jnp.int32),
        pltpu.VMEM((TPW,), jnp.int32),
        pltpu.SemaphoreType.DMA,
    ],
)
def _dispatch(x_hbm, p0_hbm, p1_hbm, xs_hbm, rows_v, i0_v, i1_v, sem):
    wid = lax.axis_index("s") * 2 + lax.axis_index("c")
    base = wid * TPW
    pltpu.sync_copy(x_hbm.at[pl.ds(base, TPW)], rows_v)
    pltpu.sync_copy(p0_hbm.at[pl.ds(base, TPW)], i0_v)
    pltpu.sync_copy(p1_hbm.at[pl.ds(base, TPW)], i1_v)
    c1 = pltpu.async_copy(rows_v, xs_hbm.at[i0_v], sem)
    c2 = pltpu.async_copy(rows_v, xs_hbm.at[i1_v], sem)
    c1.wait()
    c2.wait()


@functools.partial(
    pl.kernel,
    mesh=_SC_MESH,
    out_type=jax.ShapeDtypeStruct((T, D_OUT), jnp.float32),
    scratch_types=[
        pltpu.VMEM((TPW, D_OUT), jnp.float32),
        pltpu.VMEM((TPW, D_OUT), jnp.float32),
        pltpu.VMEM((TPW,), jnp.int32),
        pltpu.VMEM((TPW,), jnp.int32),
        pltpu.VMEM((TPW, 16), jnp.float32),
        pltpu.VMEM((TPW, 16), jnp.float32),
        pltpu.SemaphoreType.DMA,
    ],
)
def _combine(r_hbm, p0_hbm, p1_hbm, g0_hbm, g1_hbm, out_hbm,
             a_v, b_v, i0_v, i1_v, ga_v, gb_v, sem):
    wid = lax.axis_index("s") * 2 + lax.axis_index("c")
    base = wid * TPW
    pltpu.sync_copy(p0_hbm.at[pl.ds(base, TPW)], i0_v)
    pltpu.sync_copy(p1_hbm.at[pl.ds(base, TPW)], i1_v)
    pltpu.sync_copy(g0_hbm.at[pl.ds(base, TPW)], ga_v)
    pltpu.sync_copy(g1_hbm.at[pl.ds(base, TPW)], gb_v)
    c1 = pltpu.async_copy(r_hbm.at[i0_v], a_v, sem)
    c2 = pltpu.async_copy(r_hbm.at[i1_v], b_v, sem)
    c1.wait()
    c2.wait()
    eps = jnp.full((16,), EPS_OUT, jnp.float32)

    @plsc.parallel_loop(0, TPW, 1, unroll=4)
    def _row(t):
        ga = ga_v[t, :]
        gb = gb_v[t, :]
        for cidx in range(D_OUT // 16):
            sl = pl.ds(cidx * 16, 16)
            v = ga * a_v[t, sl] + gb * b_v[t, sl]
            a_v[t, sl] = jnp.where(v == 0.0, eps, v)
    pltpu.sync_copy(a_v, out_hbm.at[pl.ds(base, TPW)])


def _mlp_body(meta_ref, xs_ref, w1_ref, b1_ref, w2_ref, b2_ref, out_ref):
    b = pl.program_id(0)
    nact = meta_ref[NB, 0]

    @pl.when(b < nact)
    def _():
        xb = xs_ref[...]
        h = jnp.maximum(
            jnp.dot(xb, w1_ref[0], preferred_element_type=jnp.float32) + b1_ref[0],
            0.0,
        )
        out_ref[...] = (
            jnp.dot(h, w2_ref[0], preferred_element_type=jnp.float32) + b2_ref[0]
        )


def _grouped_mlp(meta, xs, W1, b1, W2, b2):
    grid_spec = pltpu.PrefetchScalarGridSpec(
        num_scalar_prefetch=1,
        grid=(NB,),
        in_specs=[
            pl.BlockSpec((BLK, D_IN), lambda b, m: (b, 0)),
            pl.BlockSpec((1, D_IN, H), lambda b, m: (m[b, 0], 0, 0)),
            pl.BlockSpec((1, 1, H), lambda b, m: (m[b, 0], 0, 0)),
            pl.BlockSpec((1, H, D_OUT), lambda b, m: (m[b, 0], 0, 0)),
            pl.BlockSpec((1, 1, D_OUT), lambda b, m: (m[b, 0], 0, 0)),
        ],
        out_specs=pl.BlockSpec((BLK, D_OUT), lambda b, m: (b, 0)),
    )
    return pl.pallas_call(
        _mlp_body,
        grid_spec=grid_spec,
        out_shape=jax.ShapeDtypeStruct((P, D_OUT), jnp.float32),
    )(meta, xs, W1, b1.reshape(E, 1, H), W2, b2.reshape(E, 1, D_OUT))


def kernel(x, w_gate, W1, b1, W2, b2):
    pos, g0r, g1r, meta, loss = _gating(x, w_gate)
    p0 = pos[:, 0]
    p1 = pos[:, 1]
    xs = _dispatch(x, p0, p1)
    out_sorted = _grouped_mlp(meta, xs, W1, b1, W2, b2)
    combined = _combine(out_sorted, p0, p1, g0r, g1r)
    return combined, loss.reshape(())


# speedup vs baseline: 1.4767x; 1.0755x over previous
import functools

import jax
import jax.numpy as jnp
from jax import lax
from jax.experimental import pallas as pl
from jax.experimental.pallas import tpu as pltpu
from jax.experimental.pallas import tpu_sc as plsc

D_IN = 768
D_OUT = 768
E = 8
H = 1024
KTOP = 2
T = 2048
BLK = 512
NB = (KTOP * T) // BLK + E
P = NB * BLK
CHUNK = 128
NW = 32
TPW = T // NW
EPS_OUT = 2.220446049250313e-16


def _gating_body(x_ref, wg_ref, pos_ref, g0r_ref, g1r_ref, meta_ref, loss_ref):
    x = x_ref[...]
    logits = jnp.dot(x, wg_ref[...], preferred_element_type=jnp.float32)
    lane = jax.lax.broadcasted_iota(jnp.int32, (T, E), 1)
    m1 = jnp.max(logits, axis=1, keepdims=True)
    e1 = jnp.min(jnp.where(logits == m1, lane, E), axis=1, keepdims=True)
    masked = jnp.where(lane == e1, -jnp.inf, logits)
    m2 = jnp.max(masked, axis=1, keepdims=True)
    e2 = jnp.min(jnp.where(masked == m2, lane, E), axis=1, keepdims=True)
    u = jnp.exp(m2 - m1)
    g1 = 1.0 / (1.0 + u)
    g2 = u / (1.0 + u)
    g0r_ref[...] = jnp.broadcast_to(g1, (T, 16))
    g1r_ref[...] = jnp.broadcast_to(g2, (T, 16))
    oh1 = lane == e1
    oh2 = lane == e2

    m = jnp.where(oh1 | oh2, 1.0, 0.0)
    r = jax.lax.broadcasted_iota(jnp.int32, (CHUNK, CHUNK), 0)
    c = jax.lax.broadcasted_iota(jnp.int32, (CHUNK, CHUNK), 1)
    ltri = jnp.where(r > c, 1.0, 0.0)
    chunks = []
    carry = jnp.zeros((1, E), jnp.float32)
    for j in range(T // CHUNK):
        mj = m[j * CHUNK : (j + 1) * CHUNK]
        chunks.append(jnp.dot(ltri, mj, preferred_element_type=jnp.float32) + carry)
        carry = carry + jnp.sum(mj, axis=0, keepdims=True)
    excl = jnp.concatenate(chunks, axis=0)

    counts = carry.astype(jnp.int32)
    padded = ((counts + BLK - 1) // BLK) * BLK
    inc = padded
    for sh in (1, 2, 4):
        inc = inc + jnp.concatenate(
            [jnp.zeros((1, sh), jnp.int32), inc[:, :-sh]], axis=1
        )
    seg_start = (inc - padded).astype(jnp.float32)

    def pick(oh):
        return jnp.sum(jnp.where(oh, seg_start + excl, 0.0), axis=1, keepdims=True)

    pos_ref[...] = jnp.concatenate([pick(oh1), pick(oh2)], axis=1).astype(jnp.int32)

    rowb = jax.lax.broadcasted_iota(jnp.int32, (NB, E), 0) * BLK
    blk_e = jnp.minimum(
        jnp.sum((rowb >= inc).astype(jnp.int32), axis=1, keepdims=True), E - 1
    )
    nact = inc[:, E - 1 :] // BLK
    meta_ref[...] = jnp.concatenate([blk_e, nact], axis=0)

    importance = jnp.sum(jnp.where(oh1, g1, 0.0) + jnp.where(oh2, g2, 0.0), axis=0)
    load = jnp.sum(
        oh1.astype(jnp.float32) + jnp.where(oh2 & (g2 > 0.0), 1.0, 0.0), axis=0
    )

    def cv2(v):
        mean = jnp.sum(v) / E
        var = jnp.sum((v - mean) ** 2) / (E - 1)
        return var / (mean * mean + 1e-10)

    loss_ref[...] = ((cv2(importance) + cv2(load)) * 0.01).reshape(1, 1)


def _gating(x, w_gate):
    return pl.pallas_call(
        _gating_body,
        out_shape=[
            jax.ShapeDtypeStruct((T, KTOP), jnp.int32),
            jax.ShapeDtypeStruct((T, 16), jnp.float32),
            jax.ShapeDtypeStruct((T, 16), jnp.float32),
            jax.ShapeDtypeStruct((NB + 1, 1), jnp.int32),
            jax.ShapeDtypeStruct((1, 1), jnp.float32),
        ],
    )(x, w_gate)


_SC_MESH = plsc.VectorSubcoreMesh(core_axis_name="c", subcore_axis_name="s")


@functools.partial(
    pl.kernel,
    mesh=_SC_MESH,
    out_type=jax.ShapeDtypeStruct((P, D_IN), jnp.float32),
    scratch_types=[
        pltpu.VMEM((TPW, D_IN), jnp.float32),
        pltpu.VMEM((TPW,), jnp.int32),
        pltpu.VMEM((TPW,), jnp.int32),
        pltpu.SemaphoreType.DMA,
    ],
)
def _dispatch(x_hbm, p0_hbm, p1_hbm, xs_hbm, rows_v, i0_v, i1_v, sem):
    wid = lax.axis_index("s") * 2 + lax.axis_index("c")
    base = wid * TPW
    pltpu.sync_copy(x_hbm.at[pl.ds(base, TPW)], rows_v)
    pltpu.sync_copy(p0_hbm.at[pl.ds(base, TPW)], i0_v)
    pltpu.sync_copy(p1_hbm.at[pl.ds(base, TPW)], i1_v)
    c1 = pltpu.async_copy(rows_v, xs_hbm.at[i0_v], sem)
    c2 = pltpu.async_copy(rows_v, xs_hbm.at[i1_v], sem)
    c1.wait()
    c2.wait()


@functools.partial(
    pl.kernel,
    mesh=_SC_MESH,
    out_type=jax.ShapeDtypeStruct((T, D_OUT), jnp.float32),
    scratch_types=[
        pltpu.VMEM((TPW, D_OUT), jnp.float32),
        pltpu.VMEM((TPW, D_OUT), jnp.float32),
        pltpu.VMEM((TPW,), jnp.int32),
        pltpu.VMEM((TPW,), jnp.int32),
        pltpu.VMEM((TPW, 16), jnp.float32),
        pltpu.VMEM((TPW, 16), jnp.float32),
        pltpu.SemaphoreType.DMA,
    ],
)
def _combine(r_hbm, p0_hbm, p1_hbm, g0_hbm, g1_hbm, out_hbm,
             a_v, b_v, i0_v, i1_v, ga_v, gb_v, sem):
    wid = lax.axis_index("s") * 2 + lax.axis_index("c")
    base = wid * TPW
    pltpu.sync_copy(p0_hbm.at[pl.ds(base, TPW)], i0_v)
    pltpu.sync_copy(p1_hbm.at[pl.ds(base, TPW)], i1_v)
    pltpu.sync_copy(g0_hbm.at[pl.ds(base, TPW)], ga_v)
    pltpu.sync_copy(g1_hbm.at[pl.ds(base, TPW)], gb_v)
    c1 = pltpu.async_copy(r_hbm.at[i0_v], a_v, sem)
    c2 = pltpu.async_copy(r_hbm.at[i1_v], b_v, sem)
    c1.wait()
    c2.wait()
    eps = jnp.full((16,), EPS_OUT, jnp.float32)

    @plsc.parallel_loop(0, TPW, 1, unroll=4)
    def _row(t):
        ga = ga_v[t, :]
        gb = gb_v[t, :]
        for cidx in range(D_OUT // 16):
            sl = pl.ds(cidx * 16, 16)
            v = ga * a_v[t, sl] + gb * b_v[t, sl]
            a_v[t, sl] = jnp.where(v == 0.0, eps, v)
    pltpu.sync_copy(a_v, out_hbm.at[pl.ds(base, TPW)])


def _mlp_body(meta_ref, xs_ref, w1_ref, b1_ref, w2_ref, b2_ref, out_ref):
    b = pl.program_id(0)
    nact = meta_ref[NB, 0]

    @pl.when(b < nact)
    def _():
        xb = xs_ref[...]
        h = jnp.maximum(
            jnp.dot(xb, w1_ref[0], preferred_element_type=jnp.float32) + b1_ref[0],
            0.0,
        )
        out_ref[...] = (
            jnp.dot(h, w2_ref[0], preferred_element_type=jnp.float32) + b2_ref[0]
        )


def _grouped_mlp(meta, xs, W1, b1, W2, b2):
    grid_spec = pltpu.PrefetchScalarGridSpec(
        num_scalar_prefetch=1,
        grid=(NB,),
        in_specs=[
            pl.BlockSpec((BLK, D_IN), lambda b, m: (b, 0)),
            pl.BlockSpec((1, D_IN, H), lambda b, m: (m[b, 0], 0, 0)),
            pl.BlockSpec((1, 1, H), lambda b, m: (m[b, 0], 0, 0)),
            pl.BlockSpec((1, H, D_OUT), lambda b, m: (m[b, 0], 0, 0)),
            pl.BlockSpec((1, 1, D_OUT), lambda b, m: (m[b, 0], 0, 0)),
        ],
        out_specs=pl.BlockSpec((BLK, D_OUT), lambda b, m: (b, 0)),
    )
    return pl.pallas_call(
        _mlp_body,
        grid_spec=grid_spec,
        out_shape=jax.ShapeDtypeStruct((P, D_OUT), jnp.float32),
    )(meta, xs, W1, b1.reshape(E, 1, H), W2, b2.reshape(E, 1, D_OUT))


def kernel(x, w_gate, W1, b1, W2, b2):
    pos, g0r, g1r, meta, loss = _gating(x, w_gate)
    p0 = pos[:, 0]
    p1 = pos[:, 1]
    xs = _dispatch(x, p0, p1)
    out_sorted = _grouped_mlp(meta, xs, W1, b1, W2, b2)
    combined = _combine(out_sorted, p0, p1, g0r, g1r)
    return combined, loss.reshape(())
